# Initial kernel scaffold; baseline (speedup 1.0000x reference)
import jax
import jax.numpy as jnp
from jax.experimental import pallas as pl


def kernel(features, positions, embed_table, Wmsg, W1, b1, W2, b2, element_bias, W_out):
    raise NotImplementedError("write your pallas kernel here")


# speedup vs baseline: 286.9308x; 286.9308x over previous
import jax
import jax.numpy as jnp
import numpy as np
from jax import lax
from jax.experimental import pallas as pl

_F = 32
_K = 8
_S = 9
_N = 128
_M = _K - 1
_CUTOFF = 5.0
_NITER = 3
from math import comb as _comb
_BCOEF = tuple(float(_comb(_M, i)) for i in range(_K))
_SQ3 = float(np.sqrt(3.0))


def _dot(a, b, ca, cb):
    return lax.dot_general(a, b, (((ca,), (cb,)), ((), ())),
                           preferred_element_type=jnp.float32)


def _mp_kernel(feat_ref, pos_ref, post_ref, emb_ref, bias_ref, wmsg_ref,
               w1_ref, b1_ref, w2_ref, b2_ref, wout_ref, e_ref, f_ref):
    N, K, S, F, M = _N, _K, _S, _F, _M
    f32 = jnp.float32

    pcol = pos_ref[0]
    prow = post_ref[0]
    dd = [prow[a:a + 1, :] - pcol[:, a:a + 1] for a in range(3)]
    r2 = dd[0] * dd[0] + dd[1] * dd[1] + dd[2] * dd[2] + 1e-12
    r = jnp.sqrt(r2)
    inv_r = 1.0 / r
    u = [dd[a] * inv_r for a in range(3)]

    ii = lax.broadcasted_iota(jnp.int32, (N, N), 0)
    jj = lax.broadcasted_iota(jnp.int32, (N, N), 1)
    mask = (ii != jj).astype(f32)

    xb = r / (1.0 + r)
    u5 = r * (1.0 / _CUTOFF)
    mcut = u5 < 1.0
    v2 = jnp.where(mcut, u5 * u5, 0.5)
    fc = jnp.where(mcut, jnp.exp(1.0 - 1.0 / (1.0 - v2)), 0.0)
    fcm = fc * mask

    xp = [jnp.ones_like(xb)]
    for _ in range(M):
        xp.append(xp[-1] * xb)
    om = 1.0 - xb
    op_ = [jnp.ones_like(xb)]
    for _ in range(M):
        op_.append(op_[-1] * om)
    bern = [_BCOEF[k] * xp[k] * op_[M - k] for k in range(K)]
    rad = [bern[k] * fcm for k in range(K)]

    ux, uy, uz = u[0], u[1], u[2]
    one = jnp.ones_like(ux)
    sh = [one, uy, uz, ux, _SQ3 * ux * uy, _SQ3 * uy * uz,
          0.5 * (3.0 * uz * uz - 1.0), _SQ3 * ux * uz,
          0.5 * _SQ3 * (ux * ux - uy * uy)]

    RadCat = jnp.concatenate(rad, axis=1)
    SRv = jnp.concatenate(
        [jnp.concatenate([sh[s] * rad[k] for k in range(K)], axis=1)
         for s in range(S)], axis=0)

    onehot = (feat_ref[0] == lax.broadcasted_iota(jnp.int32, (N, 128), 1)
              ).astype(f32)
    x0 = _dot(onehot, emb_ref[...], 1, 0)
    ebias = jnp.sum(onehot * bias_ref[0])

    x = jnp.concatenate([x0, jnp.zeros((S * N - N, F), f32)], axis=0)

    def make_xw(xcur, Wm):
        xs = [xcur[s * N:(s + 1) * N] for s in range(S)]
        return [jnp.concatenate([xs[s] * Wm[k:k + 1, :] for k in range(K)],
                                axis=0) for s in range(S)]

    def dxw_to_dx(dxw, Wm):
        return jnp.sum(dxw.reshape(K, N, F) * Wm[:, None, :], axis=0)

    saved = []
    xlast = None
    for i in range(_NITER):
        Wm = wmsg_ref[i]
        XW = make_xw(x, Wm)
        if i < _NITER - 1:
            A = [_dot(RadCat, XW[s], 1, 0) for s in range(S)]
            Btv = _dot(SRv, XW[0], 1, 0)
            h = x + jnp.concatenate(A, axis=0) + Btv
            t_full = _dot(h, w1_ref[i], 1, 0)
            t0 = t_full[:N] + b1_ref[i]
            t = jnp.concatenate([t0, t_full[N:]], axis=0)
            sg = jax.nn.sigmoid(t0)
            grest = (t[N:].reshape(S - 1, N, F) * sg[None]).reshape((S - 1) * N, F)
            g = jnp.concatenate([t0 * sg, grest], axis=0)
            z = _dot(g, w2_ref[i], 1, 0)
            z = jnp.concatenate([z[:N] + b2_ref[i], z[N:]], axis=0)
            saved.append((x, t))
            x = x + z
        else:
            acc = _dot(SRv[:N], XW[0], 1, 0)
            for s in range(1, S):
                acc = acc + _dot(SRv[s * N:(s + 1) * N], XW[s], 1, 0)
            h = x[:N] + acc
            t = _dot(h, w1_ref[i], 1, 0) + b1_ref[i]
            sg = jax.nn.sigmoid(t)
            z = _dot(t * sg, w2_ref[i], 1, 0) + b2_ref[i]
            saved.append((x, t))
            xlast = x[:N] + z

    woutrow = wout_ref[0]
    e_val = jnp.sum(xlast * woutrow) + ebias
    e_ref[0] = jnp.broadcast_to(jnp.reshape(e_val, (1, 1)), (1, 128))

    x_in, t2 = saved[2]
    Wm = wmsg_ref[2]
    XW = make_xw(x_in, Wm)
    dxlast = jnp.broadcast_to(woutrow, (N, F))
    gg = _dot(dxlast, w2_ref[2], 1, 1)
    sg2 = jax.nn.sigmoid(t2)
    dt = gg * (sg2 * (1.0 + t2 * (1.0 - sg2)))
    dh = _dot(dt, w1_ref[2], 1, 1)
    dy = dh
    dSRv = jnp.concatenate([_dot(dy, XW[s], 1, 1) for s in range(S)], axis=0)
    dRadCat = jnp.zeros((N, K * N), f32)
    dx_blocks = []
    for s in range(S):
        dxw = _dot(SRv[s * N:(s + 1) * N], dy, 0, 0)
        dx_blocks.append(dxw_to_dx(dxw, Wm))
    dx_blocks[0] = dx_blocks[0] + dxlast + dh
    dx = jnp.concatenate(dx_blocks, axis=0)

    for i in (1, 0):
        x_in, t = saved[i]
        Wm = wmsg_ref[i]
        XW = make_xw(x_in, Wm)
        dx_out = dx
        gg = _dot(dx_out, w2_ref[i], 1, 1)
        t0 = t[:N]
        sg = jax.nn.sigmoid(t0)
        sum_rest = jnp.sum(gg[N:].reshape(S - 1, N, F) * t[N:].reshape(S - 1, N, F),
                           axis=0)
        ds0 = gg[:N] * (sg * (1.0 + t0 * (1.0 - sg))) + sum_rest * (sg * (1.0 - sg))
        dtrest = (gg[N:].reshape(S - 1, N, F) * sg[None]).reshape((S - 1) * N, F)
        dt = jnp.concatenate([ds0, dtrest], axis=0)
        dh = _dot(dt, w1_ref[i], 1, 1)
        for s in range(S):
            dRadCat = dRadCat + _dot(dh[s * N:(s + 1) * N], XW[s], 1, 1)
        dSRv = dSRv + _dot(dh, XW[0], 1, 1)
        dxw0e = _dot(SRv, dh, 0, 0)
        dx_blocks = []
        for s in range(S):
            dxw = _dot(RadCat, dh[s * N:(s + 1) * N], 0, 0)
            if s == 0:
                dxw = dxw + dxw0e
            dx_blocks.append(dxw_to_dx(dxw, Wm))
        dx = dx_out + dh + jnp.concatenate(dx_blocks, axis=0)

    dsh = []
    drad_m = []
    for k in range(K):
        acc = dRadCat[:, k * N:(k + 1) * N]
        for s in range(S):
            acc = acc + dSRv[s * N:(s + 1) * N, k * N:(k + 1) * N] * sh[s]
        drad_m.append(acc * mask)
    for s in range(S):
        acc = dSRv[s * N:(s + 1) * N, 0:N] * rad[0]
        for k in range(1, K):
            acc = acc + dSRv[s * N:(s + 1) * N, k * N:(k + 1) * N] * rad[k]
        dsh.append(acc)

    dbern = [drad_m[k] * fc for k in range(K)]
    dfc = drad_m[0] * bern[0]
    for k in range(1, K):
        dfc = dfc + drad_m[k] * bern[k]
    dxb = jnp.zeros_like(xb)
    for k in range(K):
        d1 = k * xp[k - 1] * op_[M - k] if k > 0 else None
        d2 = (M - k) * xp[k] * op_[M - k - 1] if k < M else None
        if d1 is None:
            term = -d2
        elif d2 is None:
            term = d1
        else:
            term = d1 - d2
        dxb = dxb + dbern[k] * (_BCOEF[k] * term)
    dfc_dr = jnp.where(mcut, -fc * (2.0 * u5 * (1.0 / _CUTOFF)) / ((1.0 - v2) ** 2), 0.0)
    gr = dxb / ((1.0 + r) ** 2) + dfc * dfc_dr

    gux = dsh[3] + _SQ3 * (uy * dsh[4] + uz * dsh[7] + ux * dsh[8])
    guy = dsh[1] + _SQ3 * (ux * dsh[4] + uz * dsh[5] - uy * dsh[8])
    guz = dsh[2] + _SQ3 * (uy * dsh[5] + ux * dsh[7]) + 3.0 * uz * dsh[6]
    gu = [gux, guy, guz]
    gdu = gux * ux + guy * uy + guz * uz
    ones1 = jnp.ones((1, N), f32)
    frows = []
    for a in range(3):
        ddg = gr * u[a] + (gu[a] - gdu * u[a]) * inv_r
        rowsum = _dot(ones1, ddg, 1, 1)
        colsum = _dot(ones1, ddg, 1, 0)
        frows.append(rowsum - colsum)
    f_ref[0] = jnp.concatenate(frows, axis=0)


def _run(featcol, pos, post, emb, biasrow, Wmsg, W1, b1r, W2, b2r, woutT,
         interpret=False):
    B = featcol.shape[0]
    const = lambda *nd: (lambda b: tuple(0 for _ in range(nd[0])))
    grid = (B,)
    in_specs = [
        pl.BlockSpec((1, _N, 1), lambda b: (b, 0, 0)),
        pl.BlockSpec((1, _N, 3), lambda b: (b, 0, 0)),
        pl.BlockSpec((1, 3, _N), lambda b: (b, 0, 0)),
        pl.BlockSpec((128, _F), lambda b: (0, 0)),
        pl.BlockSpec((1, 1, 128), lambda b: (0, 0, 0)),
        pl.BlockSpec((_NITER, _K, _F), lambda b: (0, 0, 0)),
        pl.BlockSpec((_NITER, _F, _F), lambda b: (0, 0, 0)),
        pl.BlockSpec((_NITER, 1, _F), lambda b: (0, 0, 0)),
        pl.BlockSpec((_NITER, _F, _F), lambda b: (0, 0, 0)),
        pl.BlockSpec((_NITER, 1, _F), lambda b: (0, 0, 0)),
        pl.BlockSpec((1, 1, _F), lambda b: (0, 0, 0)),
    ]
    out_specs = [
        pl.BlockSpec((1, 1, 128), lambda b: (b, 0, 0)),
        pl.BlockSpec((1, 3, _N), lambda b: (b, 0, 0)),
    ]
    out_shape = [
        jax.ShapeDtypeStruct((B, 1, 128), jnp.float32),
        jax.ShapeDtypeStruct((B, 3, _N), jnp.float32),
    ]
    return pl.pallas_call(
        _mp_kernel, grid=grid, in_specs=in_specs, out_specs=out_specs,
        out_shape=out_shape, interpret=interpret,
    )(featcol, pos, post, emb, biasrow, Wmsg, W1, b1r, W2, b2r, woutT)


def kernel(features, positions, embed_table, Wmsg, W1, b1, W2, b2,
           element_bias, W_out, interpret=False):
    B, N = features.shape
    featcol = features[..., None].astype(jnp.int32)
    pos = positions.astype(jnp.float32)
    post = jnp.transpose(pos, (0, 2, 1))
    emb = jnp.zeros((128, _F), jnp.float32).at[:embed_table.shape[0]].set(embed_table)
    biasrow = jnp.zeros((1, 1, 128), jnp.float32).at[0, 0, :element_bias.shape[0]].set(element_bias)
    b1r = b1[:, None, :]
    b2r = b2[:, None, :]
    woutT = W_out.T[None]
    e_out, f_out = _run(featcol, pos, post, emb, biasrow, Wmsg, W1, b1r, W2,
                        b2r, woutT, interpret=interpret)
    E = e_out[:, 0, 0]
    forces = jnp.transpose(f_out, (0, 2, 1))
    return E, forces


# speedup vs baseline: 290.8228x; 1.0136x over previous
import jax
import jax.numpy as jnp
import numpy as np
from jax import lax
from jax.experimental import pallas as pl

_F = 32
_K = 8
_S = 9
_N = 128
_M = _K - 1
_CUTOFF = 5.0
_NITER = 3
from math import comb as _comb
_BCOEF = tuple(float(_comb(_M, i)) for i in range(_K))
_SQ3 = float(np.sqrt(3.0))


def _dot(a, b, ca, cb):
    return lax.dot_general(a, b, (((ca,), (cb,)), ((), ())),
                           preferred_element_type=jnp.float32)


def _dotb(a, b, ca, cb):
    return lax.dot_general(a.astype(jnp.bfloat16), b.astype(jnp.bfloat16),
                           (((ca,), (cb,)), ((), ())),
                           preferred_element_type=jnp.float32)


def _mp_kernel(feat_ref, pos_ref, post_ref, emb_ref, bias_ref, wmsg_ref,
               w1_ref, b1_ref, w2_ref, b2_ref, wout_ref, e_ref, f_ref):
    N, K, S, F, M = _N, _K, _S, _F, _M
    f32 = jnp.float32

    pcol = pos_ref[0]
    prow = post_ref[0]
    dd = [prow[a:a + 1, :] - pcol[:, a:a + 1] for a in range(3)]
    r2 = dd[0] * dd[0] + dd[1] * dd[1] + dd[2] * dd[2] + 1e-12
    r = jnp.sqrt(r2)
    inv_r = 1.0 / r
    u = [dd[a] * inv_r for a in range(3)]

    ii = lax.broadcasted_iota(jnp.int32, (N, N), 0)
    jj = lax.broadcasted_iota(jnp.int32, (N, N), 1)
    mask = (ii != jj).astype(f32)

    xb = r / (1.0 + r)
    u5 = r * (1.0 / _CUTOFF)
    mcut = u5 < 1.0
    v2 = jnp.where(mcut, u5 * u5, 0.5)
    fc = jnp.where(mcut, jnp.exp(1.0 - 1.0 / (1.0 - v2)), 0.0)
    fcm = fc * mask

    xp = [jnp.ones_like(xb)]
    for _ in range(M):
        xp.append(xp[-1] * xb)
    om = 1.0 - xb
    op_ = [jnp.ones_like(xb)]
    for _ in range(M):
        op_.append(op_[-1] * om)
    bern = [_BCOEF[k] * xp[k] * op_[M - k] for k in range(K)]
    rad = [bern[k] * fcm for k in range(K)]

    ux, uy, uz = u[0], u[1], u[2]
    one = jnp.ones_like(ux)
    sh = [one, uy, uz, ux, _SQ3 * ux * uy, _SQ3 * uy * uz,
          0.5 * (3.0 * uz * uz - 1.0), _SQ3 * ux * uz,
          0.5 * _SQ3 * (ux * ux - uy * uy)]

    RadCat = jnp.concatenate(rad, axis=1)
    SRv = jnp.concatenate(
        [jnp.concatenate([sh[s] * rad[k] for k in range(K)], axis=1)
         for s in range(S)], axis=0)
    RadCat_b = RadCat.astype(jnp.bfloat16)
    SRv_b = SRv.astype(jnp.bfloat16)

    onehot = (feat_ref[0] == lax.broadcasted_iota(jnp.int32, (N, 128), 1)
              ).astype(f32)
    x0 = _dot(onehot, emb_ref[...], 1, 0)
    ebias = jnp.sum(onehot * bias_ref[0])

    x = jnp.concatenate([x0, jnp.zeros((S * N - N, F), f32)], axis=0)

    def make_xw(xcur, Wm):
        xs = [xcur[s * N:(s + 1) * N] for s in range(S)]
        return [jnp.concatenate([(xs[s] * Wm[k:k + 1, :]).astype(jnp.bfloat16)
                                 for k in range(K)], axis=0)
                for s in range(S)]

    def dxw_to_dx(dxw, Wm):
        acc = dxw[:N] * Wm[0:1, :]
        for k in range(1, K):
            acc = acc + dxw[k * N:(k + 1) * N] * Wm[k:k + 1, :]
        return acc

    saved = []
    xlast = None
    for i in range(_NITER):
        Wm = wmsg_ref[i]
        XW = make_xw(x, Wm)
        if i < _NITER - 1:
            A = [_dotb(RadCat_b, XW[s], 1, 0) for s in range(S)]
            Btv = _dotb(SRv_b, XW[0], 1, 0)
            h = x + jnp.concatenate(A, axis=0) + Btv
            t_full = _dot(h, w1_ref[i], 1, 0)
            t0 = t_full[:N] + b1_ref[i]
            t = jnp.concatenate([t0, t_full[N:]], axis=0)
            sg = jax.nn.sigmoid(t0)
            g = jnp.concatenate(
                [t0 * sg] + [t_full[s * N:(s + 1) * N] * sg for s in range(1, S)],
                axis=0)
            z = _dot(g, w2_ref[i], 1, 0)
            z = jnp.concatenate([z[:N] + b2_ref[i], z[N:]], axis=0)
            saved.append((x, t, XW))
            x = x + z
        else:
            acc = _dotb(SRv_b[:N], XW[0], 1, 0)
            for s in range(1, S):
                acc = acc + _dotb(SRv_b[s * N:(s + 1) * N], XW[s], 1, 0)
            h = x[:N] + acc
            t = _dot(h, w1_ref[i], 1, 0) + b1_ref[i]
            sg = jax.nn.sigmoid(t)
            z = _dot(t * sg, w2_ref[i], 1, 0) + b2_ref[i]
            saved.append((x, t, XW))
            xlast = x[:N] + z

    woutrow = wout_ref[0]
    e_val = jnp.sum(xlast * woutrow) + ebias
    e_ref[0] = jnp.broadcast_to(jnp.reshape(e_val, (1, 1)), (1, 128))

    x_in, t2, XW = saved[2]
    Wm = wmsg_ref[2]
    dxlast = jnp.broadcast_to(woutrow, (N, F))
    gg = _dot(dxlast, w2_ref[2], 1, 1)
    sg2 = jax.nn.sigmoid(t2)
    dt = gg * (sg2 * (1.0 + t2 * (1.0 - sg2)))
    dh = _dot(dt, w1_ref[2], 1, 1)
    dy = dh.astype(jnp.bfloat16)
    dSRv = jnp.concatenate([_dotb(dy, XW[s], 1, 1) for s in range(S)], axis=0)
    dRadCat = jnp.zeros((N, K * N), f32)
    dx_blocks = []
    for s in range(S):
        dxw = _dotb(SRv_b[s * N:(s + 1) * N], dy, 0, 0)
        dx_blocks.append(dxw_to_dx(dxw, Wm))
    dx_blocks[0] = dx_blocks[0] + dxlast + dh
    dx = jnp.concatenate(dx_blocks, axis=0)

    for i in (1, 0):
        x_in, t, XW = saved[i]
        Wm = wmsg_ref[i]
        dx_out = dx
        gg = _dot(dx_out, w2_ref[i], 1, 1)
        t0 = t[:N]
        sg = jax.nn.sigmoid(t0)
        sum_rest = gg[N:2 * N] * t[N:2 * N]
        for s in range(2, S):
            sum_rest = sum_rest + gg[s * N:(s + 1) * N] * t[s * N:(s + 1) * N]
        ds0 = gg[:N] * (sg * (1.0 + t0 * (1.0 - sg))) + sum_rest * (sg * (1.0 - sg))
        dt = jnp.concatenate(
            [ds0] + [gg[s * N:(s + 1) * N] * sg for s in range(1, S)], axis=0)
        dh = _dot(dt, w1_ref[i], 1, 1)
        dh_b = dh.astype(jnp.bfloat16)
        for s in range(S):
            dRadCat = dRadCat + _dotb(dh_b[s * N:(s + 1) * N], XW[s], 1, 1)
        dSRv = dSRv + _dotb(dh_b, XW[0], 1, 1)
        dxw0e = _dotb(SRv_b, dh_b, 0, 0)
        dx_blocks = []
        for s in range(S):
            dxw = _dotb(RadCat_b, dh_b[s * N:(s + 1) * N], 0, 0)
            if s == 0:
                dxw = dxw + dxw0e
            dx_blocks.append(dxw_to_dx(dxw, Wm))
        dx = dx_out + dh + jnp.concatenate(dx_blocks, axis=0)

    dsh = []
    drad_m = []
    for k in range(K):
        acc = dRadCat[:, k * N:(k + 1) * N]
        for s in range(S):
            acc = acc + dSRv[s * N:(s + 1) * N, k * N:(k + 1) * N] * sh[s]
        drad_m.append(acc * mask)
    for s in range(S):
        acc = dSRv[s * N:(s + 1) * N, 0:N] * rad[0]
        for k in range(1, K):
            acc = acc + dSRv[s * N:(s + 1) * N, k * N:(k + 1) * N] * rad[k]
        dsh.append(acc)

    dbern = [drad_m[k] * fc for k in range(K)]
    dfc = drad_m[0] * bern[0]
    for k in range(1, K):
        dfc = dfc + drad_m[k] * bern[k]
    dxb = jnp.zeros_like(xb)
    for k in range(K):
        d1 = k * xp[k - 1] * op_[M - k] if k > 0 else None
        d2 = (M - k) * xp[k] * op_[M - k - 1] if k < M else None
        if d1 is None:
            term = -d2
        elif d2 is None:
            term = d1
        else:
            term = d1 - d2
        dxb = dxb + dbern[k] * (_BCOEF[k] * term)
    dfc_dr = jnp.where(mcut, -fc * (2.0 * u5 * (1.0 / _CUTOFF)) / ((1.0 - v2) ** 2), 0.0)
    gr = dxb / ((1.0 + r) ** 2) + dfc * dfc_dr

    gux = dsh[3] + _SQ3 * (uy * dsh[4] + uz * dsh[7] + ux * dsh[8])
    guy = dsh[1] + _SQ3 * (ux * dsh[4] + uz * dsh[5] - uy * dsh[8])
    guz = dsh[2] + _SQ3 * (uy * dsh[5] + ux * dsh[7]) + 3.0 * uz * dsh[6]
    gu = [gux, guy, guz]
    gdu = gux * ux + guy * uy + guz * uz
    ones1 = jnp.ones((1, N), f32)
    frows = []
    for a in range(3):
        ddg = gr * u[a] + (gu[a] - gdu * u[a]) * inv_r
        rowsum = _dot(ones1, ddg, 1, 1)
        colsum = _dot(ones1, ddg, 1, 0)
        frows.append(rowsum - colsum)
    f_ref[0] = jnp.concatenate(frows, axis=0)


def _run(featcol, pos, post, emb, biasrow, Wmsg, W1, b1r, W2, b2r, woutT,
         interpret=False):
    B = featcol.shape[0]
    const = lambda *nd: (lambda b: tuple(0 for _ in range(nd[0])))
    grid = (B,)
    in_specs = [
        pl.BlockSpec((1, _N, 1), lambda b: (b, 0, 0)),
        pl.BlockSpec((1, _N, 3), lambda b: (b, 0, 0)),
        pl.BlockSpec((1, 3, _N), lambda b: (b, 0, 0)),
        pl.BlockSpec((128, _F), lambda b: (0, 0)),
        pl.BlockSpec((1, 1, 128), lambda b: (0, 0, 0)),
        pl.BlockSpec((_NITER, _K, _F), lambda b: (0, 0, 0)),
        pl.BlockSpec((_NITER, _F, _F), lambda b: (0, 0, 0)),
        pl.BlockSpec((_NITER, 1, _F), lambda b: (0, 0, 0)),
        pl.BlockSpec((_NITER, _F, _F), lambda b: (0, 0, 0)),
        pl.BlockSpec((_NITER, 1, _F), lambda b: (0, 0, 0)),
        pl.BlockSpec((1, 1, _F), lambda b: (0, 0, 0)),
    ]
    out_specs = [
        pl.BlockSpec((1, 1, 128), lambda b: (b, 0, 0)),
        pl.BlockSpec((1, 3, _N), lambda b: (b, 0, 0)),
    ]
    out_shape = [
        jax.ShapeDtypeStruct((B, 1, 128), jnp.float32),
        jax.ShapeDtypeStruct((B, 3, _N), jnp.float32),
    ]
    return pl.pallas_call(
        _mp_kernel, grid=grid, in_specs=in_specs, out_specs=out_specs,
        out_shape=out_shape, interpret=interpret,
    )(featcol, pos, post, emb, biasrow, Wmsg, W1, b1r, W2, b2r, woutT)


def kernel(features, positions, embed_table, Wmsg, W1, b1, W2, b2,
           element_bias, W_out, interpret=False):
    B, N = features.shape
    featcol = features[..., None].astype(jnp.int32)
    pos = positions.astype(jnp.float32)
    post = jnp.transpose(pos, (0, 2, 1))
    emb = jnp.zeros((128, _F), jnp.float32).at[:embed_table.shape[0]].set(embed_table)
    biasrow = jnp.zeros((1, 1, 128), jnp.float32).at[0, 0, :element_bias.shape[0]].set(element_bias)
    b1r = b1[:, None, :]
    b2r = b2[:, None, :]
    woutT = W_out.T[None]
    e_out, f_out = _run(featcol, pos, post, emb, biasrow, Wmsg, W1, b1r, W2,
                        b2r, woutT, interpret=interpret)
    E = e_out[:, 0, 0]
    forces = jnp.transpose(f_out, (0, 2, 1))
    return E, forces
